# baseline (speedup 1.0000x reference)
import functools

import jax
import jax.numpy as jnp
from jax import lax
from jax.experimental import pallas as pl
from jax.experimental.pallas import tpu as pltpu
from jax.experimental.pallas import tpu_sc as plsc

B = 1024
N = 100000
K = 30
KP = 32
NEG = -1e30
CAP = 1024
G = 22
GE = G * 16
TRIG = CAP - GE - 16
NMAIN = 99968
TW = N - NMAIN
CHW = 5632
NFULL = 17
LASTW = NMAIN - NFULL * CHW
NGF = CHW // GE
NGL = LASTW // GE
NC = 2
NS = 16
NW = NC * NS
RPW = B // NW
NOCT = RPW // 8


NTC = 782
TCW = B * 128


def _flat_index(row, col):
    return (col >> 7) * TCW + row * 128 + (col & 127)


def _sc_topk_body(tgt_hbm, tail_hbm, tt_hbm, ti_hbm,
                  chunk_v, tail_v, cand_v, cand_i, val_v, idx_v,
                  thr_v, cnt_s, sem):
    wid = lax.axis_index("s") * NC + lax.axis_index("c")
    row0 = wid * RPW
    iota = lax.iota(jnp.int32, 16)
    neg16 = jnp.full((16,), NEG, jnp.float32)

    def sort16(v, p):
        sv, sp = plsc.sort_key_val(v, p, descending=True)
        return sv, sp

    def merge_top16(av, ap, bv, bp):
        bvr = lax.rev(bv, (0,))
        bpr = lax.rev(bp, (0,))
        ta = av >= bvr
        return sort16(jnp.where(ta, av, bvr), jnp.where(ta, ap, bpr))

    def round_top16(r8, nvr):
        bv, bp = sort16(cand_v[r8, pl.ds(0, 16)], iota)

        def rbody(j, carry):
            sv, sp = sort16(cand_v[r8, pl.ds(j * 16, 16)], iota + j * 16)
            return merge_top16(carry[0], carry[1], sv, sp)

        return lax.fori_loop(1, nvr, rbody, (bv, bp))

    def select_top32(r8):
        cnt = cnt_s[r8]
        cand_v[r8, pl.ds(cnt, 16)] = neg16
        nvr = cnt // 16 + 1
        r8v = jnp.full((16,), r8, jnp.int32)
        v1, p1 = round_top16(r8, nvr)
        saved = plsc.load_gather(cand_v, [r8v, p1])
        plsc.store_scatter(cand_v, [r8v, p1], neg16)
        v2, p2 = round_top16(r8, nvr)
        plsc.store_scatter(cand_v, [r8v, p1], saved)
        i1 = plsc.load_gather(cand_i, [r8v, p1])
        i2 = plsc.load_gather(cand_i, [r8v, p2])
        return v1, i1, v2, i2

    def append(r8, vals, cols, mask):
        cnt = cnt_s[r8]
        plsc.store_compressed(cand_v.at[r8, pl.ds(cnt, 16)], vals, mask=mask)
        plsc.store_compressed(cand_i.at[r8, pl.ds(cnt, 16)], cols, mask=mask)
        cnt_s[r8] = cnt + jnp.sum(mask.astype(jnp.int32))

    def compact(r8):
        @pl.when(cnt_s[r8] > TRIG)
        def _():
            v1, i1, v2, i2 = select_top32(r8)
            cand_v[r8, pl.ds(0, 16)] = v1
            cand_v[r8, pl.ds(16, 16)] = v2
            cand_i[r8, pl.ds(0, 16)] = i1
            cand_i[r8, pl.ds(16, 16)] = i2
            cnt_s[r8] = KP
            thr_v[r8, :] = plsc.load_gather(
                cand_v,
                [jnp.full((16,), r8, jnp.int32),
                 jnp.full((16,), 31, jnp.int32)])

    def scan_rows(s, cbase, ngroups):
        def row_scan(r8, _):
            r8v = jnp.full((16,), r8, jnp.int32)

            def group_body(g, _):
                base = g * GE
                thr = thr_v[r8, :]
                off = jnp.full((16,), cnt_s[r8], jnp.int32)
                colbase = cbase + base
                for j in range(G):
                    v = chunk_v[s, r8, pl.ds(base + j * 16, 16)]
                    m = v > thr
                    pc = plsc.cumsum(m.astype(jnp.int32))
                    pos = off + pc - 1
                    plsc.store_scatter(cand_v, [r8v, pos], v, mask=m)
                    plsc.store_scatter(cand_i, [r8v, pos],
                                       colbase + j * 16 + iota, mask=m)
                    off = off + plsc.all_reduce_population_count(m)
                cnt_s[r8] = jnp.max(off)
                compact(r8)
                return 0

            lax.fori_loop(0, ngroups, group_body, 0)
            return 0

        lax.fori_loop(0, 8, row_scan, 0)

    def octet_body(o, _):
        row8 = pl.multiple_of(row0 + o * 8, 8)

        def full_copy(c, s):
            return pltpu.make_async_copy(
                tgt_hbm.at[pl.ds(row8, 8),
                           pl.ds(pl.multiple_of(c * CHW, 128), CHW)],
                chunk_v.at[s, :, pl.ds(0, CHW)], sem)

        def last_copy(s):
            return pltpu.make_async_copy(
                tgt_hbm.at[pl.ds(row8, 8), pl.ds(NFULL * CHW, LASTW)],
                chunk_v.at[s, :, pl.ds(0, LASTW)], sem)

        def rbody(r8, _):
            thr_v[r8, :] = neg16
            cnt_s[r8] = 0
            return 0

        lax.fori_loop(0, 8, rbody, 0)

        full_copy(0, 0).start()
        pltpu.sync_copy(tail_hbm.at[pl.ds(row8 * TW, 8 * TW)], tail_v)

        def chunk_body(c, _):
            s = lax.rem(c, 2)
            full_copy(c, s).wait()

            @pl.when(c + 1 < NFULL)
            def _():
                full_copy(c + 1, 1 - s).start()

            @pl.when(c + 1 == NFULL)
            def _():
                last_copy(1 - s).start()

            scan_rows(s, c * CHW, NGF)
            return 0

        lax.fori_loop(0, NFULL, chunk_body, 0)
        st = NFULL % 2
        last_copy(st).wait()
        scan_rows(st, NFULL * CHW, NGL)

        def tail_scan(r8, _):
            v0 = tail_v[pl.ds(r8 * TW, 16)]
            v1 = tail_v[pl.ds(r8 * TW + 16, 16)]
            thr = thr_v[r8, :]
            hit = jnp.any(jnp.maximum(v0, v1) > thr)

            @pl.when(hit)
            def _():
                append(r8, v0, NMAIN + iota, v0 > thr)
                append(r8, v1, NMAIN + 16 + iota, v1 > thr)
                compact(r8)

            return 0

        lax.fori_loop(0, 8, tail_scan, 0)

        def out_body(r8, _):
            v1, i1, v2, i2 = select_top32(r8)
            rl = o * 8 + r8
            val_v[pl.ds(rl * KP, 16)] = v1
            val_v[pl.ds(rl * KP + 16, 16)] = v2
            row = row8 + r8
            idx_v[pl.ds(rl * KP, 16)] = _flat_index(row, i1)
            idx_v[pl.ds(rl * KP + 16, 16)] = _flat_index(row, i2)
            return 0

        lax.fori_loop(0, 8, out_body, 0)
        return 0

    lax.fori_loop(0, NOCT, octet_body, 0)

    out0 = wid * RPW * KP
    pltpu.sync_copy(val_v, tt_hbm.at[pl.ds(out0, RPW * KP)])
    pltpu.sync_copy(idx_v, ti_hbm.at[pl.ds(out0, RPW * KP)])


_sc_topk = functools.partial(
    pl.kernel,
    out_type=(jax.ShapeDtypeStruct((B * KP,), jnp.float32),
              jax.ShapeDtypeStruct((B * KP,), jnp.int32)),
    mesh=plsc.VectorSubcoreMesh(core_axis_name="c", subcore_axis_name="s"),
    compiler_params=pltpu.CompilerParams(needs_layout_passes=False),
    scratch_types=(
        pltpu.VMEM((2, 8, CHW), jnp.float32),
        pltpu.VMEM((8 * TW,), jnp.float32),
        pltpu.VMEM((8, CAP), jnp.float32),
        pltpu.VMEM((8, CAP), jnp.int32),
        pltpu.VMEM((RPW * KP,), jnp.float32),
        pltpu.VMEM((RPW * KP,), jnp.int32),
        pltpu.VMEM((8, 16), jnp.float32),
        pltpu.SMEM((8,), jnp.int32),
        pltpu.SemaphoreType.DMA,
    ),
)(_sc_topk_body)


def _sc_gather_body(lin_hbm, ti_hbm, tl_hbm, idx_v, lgt_v, sem, gsem):
    wid = lax.axis_index("s") * NC + lax.axis_index("c")
    base = wid * RPW * KP
    pltpu.async_copy(ti_hbm.at[pl.ds(base, RPW * KP)], idx_v, sem).wait()
    for q in range(RPW * KP // 128):
        pltpu.async_copy(lin_hbm.at[idx_v.at[pl.ds(q * 128, 128)]],
                         lgt_v.at[pl.ds(q * 128, 128)], gsem).wait()
    pltpu.sync_copy(lgt_v, tl_hbm.at[pl.ds(base, RPW * KP)])


_sc_gather = functools.partial(
    pl.kernel,
    out_type=jax.ShapeDtypeStruct((B * KP,), jnp.float32),
    mesh=plsc.VectorSubcoreMesh(core_axis_name="c", subcore_axis_name="s"),
    compiler_params=pltpu.CompilerParams(needs_layout_passes=False),
    scratch_types=(
        pltpu.VMEM((RPW * KP,), jnp.int32),
        pltpu.VMEM((RPW * KP,), jnp.float32),
        pltpu.SemaphoreType.DMA,
        pltpu.SemaphoreType.DMA,
    ),
)(_sc_gather_body)


def _lin_body(x_ref, o_ref):
    o_ref[...] = x_ref[...].reshape(TCW)


_lin_logits = pl.pallas_call(
    _lin_body,
    grid=(NTC,),
    in_specs=[pl.BlockSpec((B, 128), lambda i: (0, i))],
    out_specs=pl.BlockSpec((TCW,), lambda i: (i,)),
    out_shape=jax.ShapeDtypeStruct((NTC * TCW,), jnp.float32),
)


def _loss_body(t_ref, l_ref, o_ref):
    t = t_ref[...]
    l = l_ref[...]
    mask = lax.broadcasted_iota(jnp.int32, (B, KP), 1) < K
    t = jnp.where(mask, t, NEG)
    l = jnp.where(mask, l, NEG)
    tm = jnp.max(t, axis=1, keepdims=True)
    lm = jnp.max(l, axis=1, keepdims=True)
    te = jnp.exp(t - tm)
    le = jnp.exp(l - lm)
    ts = jnp.sum(jnp.where(mask, te, 0.0), axis=1, keepdims=True)
    ls = jnp.sum(jnp.where(mask, le, 0.0), axis=1, keepdims=True)
    pt = te / ts
    diff = (t - tm) - jnp.log(ts) - (l - lm) + jnp.log(ls)
    pw = jnp.where(mask, pt * diff, 0.0)
    o_ref[...] = jnp.broadcast_to(jnp.sum(pw) * (1.0 / B), (1, 1))


def kernel(logits, targets):
    tail = targets[:, NMAIN:].reshape(-1)
    tt_flat, ti_flat = _sc_topk(targets, tail)
    lin = _lin_logits(logits)
    tl_flat = _sc_gather(lin, ti_flat)
    tt = tt_flat.reshape(B, KP)
    tl = tl_flat.reshape(B, KP)
    loss = pl.pallas_call(
        _loss_body,
        out_shape=jax.ShapeDtypeStruct((1, 1), jnp.float32),
    )(tt, tl)
    return loss[0, 0]


# speedup vs baseline: 2.6215x; 2.6215x over previous
import functools

import jax
import jax.numpy as jnp
from jax import lax
from jax.experimental import pallas as pl
from jax.experimental.pallas import tpu as pltpu
from jax.experimental.pallas import tpu_sc as plsc

B = 1024
N = 100000
K = 30
KP = 32
NEG = -1e30
CAP = 1024
G = 22
GE = G * 16
TRIG = CAP - GE - 16
NMAIN = 99968
TW = N - NMAIN
CHW = 5632
NFULL = 17
LASTW = NMAIN - NFULL * CHW
NGF = CHW // GE
NGL = LASTW // GE
NC = 2
NS = 16
NW = NC * NS
RPW = B // NW
NOCT = RPW // 8


NTC = 782
TCW = B * 128


def _flat_index(row, col):
    return (col >> 7) * TCW + row * 128 + (col & 127)


def _sc_topk_body(tgt_hbm, tail_hbm, tt_hbm, ti_hbm,
                  chunk_v, tail_v, cand_v, cand_i, val_v, idx_v,
                  thr_v, cnt_s, sem):
    wid = lax.axis_index("s") * NC + lax.axis_index("c")
    row0 = wid * RPW
    iota = lax.iota(jnp.int32, 16)
    neg16 = jnp.full((16,), NEG, jnp.float32)

    def sort16(v, p):
        sv, sp = plsc.sort_key_val(v, p, descending=True)
        return sv, sp

    def merge_top16(av, ap, bv, bp):
        bvr = lax.rev(bv, (0,))
        bpr = lax.rev(bp, (0,))
        ta = av >= bvr
        return sort16(jnp.where(ta, av, bvr), jnp.where(ta, ap, bpr))

    def round_top16(rbase, nvr):
        bv, bp = sort16(cand_v[pl.ds(rbase, 16)], iota)

        def rbody(j, carry):
            sv, sp = sort16(cand_v[pl.ds(rbase + j * 16, 16)],
                            iota + j * 16)
            return merge_top16(carry[0], carry[1], sv, sp)

        return lax.fori_loop(1, nvr, rbody, (bv, bp))

    def select_top32(r8):
        rbase = r8 * CAP
        cnt = cnt_s[r8]
        cand_v[pl.ds(rbase + cnt, 16)] = neg16
        nvr = cnt // 16 + 1
        v1, p1 = round_top16(rbase, nvr)
        saved = plsc.load_gather(cand_v, [rbase + p1])
        plsc.store_scatter(cand_v, [rbase + p1], neg16)
        v2, p2 = round_top16(rbase, nvr)
        plsc.store_scatter(cand_v, [rbase + p1], saved)
        i1 = plsc.load_gather(cand_i, [rbase + p1])
        i2 = plsc.load_gather(cand_i, [rbase + p2])
        return v1, i1, v2, i2

    def append(r8, vals, cols, mask):
        cnt = r8 * CAP + cnt_s[r8]
        plsc.store_compressed(cand_v.at[pl.ds(cnt, 16)], vals, mask=mask)
        plsc.store_compressed(cand_i.at[pl.ds(cnt, 16)], cols, mask=mask)
        cnt_s[r8] = cnt_s[r8] + jnp.sum(mask.astype(jnp.int32))

    def compact(r8):
        @pl.when(cnt_s[r8] > TRIG)
        def _():
            rbase = r8 * CAP
            v1, i1, v2, i2 = select_top32(r8)
            cand_v[pl.ds(rbase, 16)] = v1
            cand_v[pl.ds(rbase + 16, 16)] = v2
            cand_i[pl.ds(rbase, 16)] = i1
            cand_i[pl.ds(rbase + 16, 16)] = i2
            cnt_s[r8] = KP
            thr_v[r8, :] = plsc.load_gather(
                cand_v, [jnp.full((16,), 31, jnp.int32) + rbase])

    def scan_rows(s, cbase, ngroups):
        def row_scan(r8, _):
            def group_body(g, _):
                base = g * GE
                thr = thr_v[r8, :]
                colbase = cbase + base
                vs, ms, pcs, cts = [], [], [], []
                for j in range(G):
                    v = chunk_v[s, r8, pl.ds(base + j * 16, 16)]
                    m = v > thr
                    vs.append(v)
                    ms.append(m)
                    pcs.append(plsc.cumsum(m.astype(jnp.int32)))
                    cts.append(plsc.all_reduce_population_count(m))
                off = jnp.full((16,), r8 * CAP + cnt_s[r8] - 1, jnp.int32)
                offs = []
                for j in range(G):
                    offs.append(off)
                    off = off + cts[j]
                for j in range(G):
                    pos = offs[j] + pcs[j]
                    plsc.store_scatter(cand_v, [pos], vs[j], mask=ms[j])
                    plsc.store_scatter(cand_i, [pos],
                                       colbase + j * 16 + iota, mask=ms[j])
                cnt_s[r8] = jnp.max(off) + 1 - r8 * CAP
                compact(r8)
                return 0

            lax.fori_loop(0, ngroups, group_body, 0)
            return 0

        lax.fori_loop(0, 8, row_scan, 0)

    def octet_body(o, _):
        row8 = pl.multiple_of(row0 + o * 8, 8)

        def full_copy(c, s):
            return pltpu.make_async_copy(
                tgt_hbm.at[pl.ds(row8, 8),
                           pl.ds(pl.multiple_of(c * CHW, 128), CHW)],
                chunk_v.at[s, :, pl.ds(0, CHW)], sem)

        def last_copy(s):
            return pltpu.make_async_copy(
                tgt_hbm.at[pl.ds(row8, 8), pl.ds(NFULL * CHW, LASTW)],
                chunk_v.at[s, :, pl.ds(0, LASTW)], sem)

        def rbody(r8, _):
            thr_v[r8, :] = neg16
            cnt_s[r8] = 0
            return 0

        lax.fori_loop(0, 8, rbody, 0)

        full_copy(0, 0).start()
        pltpu.sync_copy(tail_hbm.at[pl.ds(row8 * TW, 8 * TW)], tail_v)

        def chunk_body(c, _):
            s = lax.rem(c, 2)
            full_copy(c, s).wait()

            @pl.when(c + 1 < NFULL)
            def _():
                full_copy(c + 1, 1 - s).start()

            @pl.when(c + 1 == NFULL)
            def _():
                last_copy(1 - s).start()

            scan_rows(s, c * CHW, NGF)
            return 0

        lax.fori_loop(0, NFULL, chunk_body, 0)
        st = NFULL % 2
        last_copy(st).wait()
        scan_rows(st, NFULL * CHW, NGL)

        def tail_scan(r8, _):
            v0 = tail_v[pl.ds(r8 * TW, 16)]
            v1 = tail_v[pl.ds(r8 * TW + 16, 16)]
            thr = thr_v[r8, :]
            hit = jnp.any(jnp.maximum(v0, v1) > thr)

            @pl.when(hit)
            def _():
                append(r8, v0, NMAIN + iota, v0 > thr)
                append(r8, v1, NMAIN + 16 + iota, v1 > thr)
                compact(r8)

            return 0

        lax.fori_loop(0, 8, tail_scan, 0)

        def out_body(r8, _):
            v1, i1, v2, i2 = select_top32(r8)
            rl = o * 8 + r8
            val_v[pl.ds(rl * KP, 16)] = v1
            val_v[pl.ds(rl * KP + 16, 16)] = v2
            row = row8 + r8
            idx_v[pl.ds(rl * KP, 16)] = _flat_index(row, i1)
            idx_v[pl.ds(rl * KP + 16, 16)] = _flat_index(row, i2)
            return 0

        lax.fori_loop(0, 8, out_body, 0)
        return 0

    lax.fori_loop(0, NOCT, octet_body, 0)

    out0 = wid * RPW * KP
    pltpu.sync_copy(val_v, tt_hbm.at[pl.ds(out0, RPW * KP)])
    pltpu.sync_copy(idx_v, ti_hbm.at[pl.ds(out0, RPW * KP)])


_sc_topk = functools.partial(
    pl.kernel,
    out_type=(jax.ShapeDtypeStruct((B * KP,), jnp.float32),
              jax.ShapeDtypeStruct((B * KP,), jnp.int32)),
    mesh=plsc.VectorSubcoreMesh(core_axis_name="c", subcore_axis_name="s"),
    compiler_params=pltpu.CompilerParams(needs_layout_passes=False),
    scratch_types=(
        pltpu.VMEM((2, 8, CHW), jnp.float32),
        pltpu.VMEM((8 * TW,), jnp.float32),
        pltpu.VMEM((8 * CAP,), jnp.float32),
        pltpu.VMEM((8 * CAP,), jnp.int32),
        pltpu.VMEM((RPW * KP,), jnp.float32),
        pltpu.VMEM((RPW * KP,), jnp.int32),
        pltpu.VMEM((8, 16), jnp.float32),
        pltpu.SMEM((8,), jnp.int32),
        pltpu.SemaphoreType.DMA,
    ),
)(_sc_topk_body)


def _sc_gather_body(lin_hbm, ti_hbm, tl_hbm, idx_v, lgt_v, sem, gsem):
    wid = lax.axis_index("s") * NC + lax.axis_index("c")
    base = wid * RPW * KP
    pltpu.async_copy(ti_hbm.at[pl.ds(base, RPW * KP)], idx_v, sem).wait()
    for q in range(RPW * KP // 128):
        pltpu.async_copy(lin_hbm.at[idx_v.at[pl.ds(q * 128, 128)]],
                         lgt_v.at[pl.ds(q * 128, 128)], gsem).wait()
    pltpu.sync_copy(lgt_v, tl_hbm.at[pl.ds(base, RPW * KP)])


_sc_gather = functools.partial(
    pl.kernel,
    out_type=jax.ShapeDtypeStruct((B * KP,), jnp.float32),
    mesh=plsc.VectorSubcoreMesh(core_axis_name="c", subcore_axis_name="s"),
    compiler_params=pltpu.CompilerParams(needs_layout_passes=False),
    scratch_types=(
        pltpu.VMEM((RPW * KP,), jnp.int32),
        pltpu.VMEM((RPW * KP,), jnp.float32),
        pltpu.SemaphoreType.DMA,
        pltpu.SemaphoreType.DMA,
    ),
)(_sc_gather_body)


def _lin_body(x_ref, o_ref):
    o_ref[...] = x_ref[...].reshape(TCW)


_lin_logits = pl.pallas_call(
    _lin_body,
    grid=(NTC,),
    in_specs=[pl.BlockSpec((B, 128), lambda i: (0, i))],
    out_specs=pl.BlockSpec((TCW,), lambda i: (i,)),
    out_shape=jax.ShapeDtypeStruct((NTC * TCW,), jnp.float32),
)


def _loss_body(t_ref, l_ref, o_ref):
    t = t_ref[...]
    l = l_ref[...]
    mask = lax.broadcasted_iota(jnp.int32, (B, KP), 1) < K
    t = jnp.where(mask, t, NEG)
    l = jnp.where(mask, l, NEG)
    tm = jnp.max(t, axis=1, keepdims=True)
    lm = jnp.max(l, axis=1, keepdims=True)
    te = jnp.exp(t - tm)
    le = jnp.exp(l - lm)
    ts = jnp.sum(jnp.where(mask, te, 0.0), axis=1, keepdims=True)
    ls = jnp.sum(jnp.where(mask, le, 0.0), axis=1, keepdims=True)
    pt = te / ts
    diff = (t - tm) - jnp.log(ts) - (l - lm) + jnp.log(ls)
    pw = jnp.where(mask, pt * diff, 0.0)
    o_ref[...] = jnp.broadcast_to(jnp.sum(pw) * (1.0 / B), (1, 1))


def kernel(logits, targets):
    tail = targets[:, NMAIN:].reshape(-1)
    tt_flat, ti_flat = _sc_topk(targets, tail)
    lin = _lin_logits(logits)
    tl_flat = _sc_gather(lin, ti_flat)
    tt = tt_flat.reshape(B, KP)
    tl = tl_flat.reshape(B, KP)
    loss = pl.pallas_call(
        _loss_body,
        out_shape=jax.ShapeDtypeStruct((1, 1), jnp.float32),
    )(tt, tl)
    return loss[0, 0]


# speedup vs baseline: 3.2378x; 1.2351x over previous
import functools

import jax
import jax.numpy as jnp
from jax import lax
from jax.experimental import pallas as pl
from jax.experimental.pallas import tpu as pltpu
from jax.experimental.pallas import tpu_sc as plsc

B = 1024
N = 100000
K = 30
KP = 32
NEG = -1e30
CAP = 1024
G = 22
GE = G * 16
TRIG = CAP - GE - 16
NMAIN = 99968
TW = N - NMAIN
CHW = 5632
NFULL = 17
LASTW = NMAIN - NFULL * CHW
NGF = CHW // GE
NGL = LASTW // GE
NC = 2
NS = 16
NW = NC * NS
RPW = B // NW
NOCT = RPW // 8


NTC = 782
TCW = B * 128


def _flat_index(row, col):
    return (col >> 7) * TCW + row * 128 + (col & 127)


def _sc_topk_body(tgt_hbm, tail_hbm, tt_hbm, ti_hbm,
                  chunk_v, tail_v, cand_v, cand_i, val_v, idx_v,
                  thr_v, cnt_s, sem):
    wid = lax.axis_index("s") * NC + lax.axis_index("c")
    row0 = wid * RPW
    iota = lax.iota(jnp.int32, 16)
    neg16 = jnp.full((16,), NEG, jnp.float32)

    def sort16(v, p):
        sv, sp = plsc.sort_key_val(v, p, descending=True)
        return sv, sp

    def merge_top16(av, ap, bv, bp):
        bvr = lax.rev(bv, (0,))
        bpr = lax.rev(bp, (0,))
        ta = av >= bvr
        return sort16(jnp.where(ta, av, bvr), jnp.where(ta, ap, bpr))

    def round_top16(rbase, nvr):
        bv, bp = sort16(cand_v[pl.ds(rbase, 16)], iota)

        def rbody(j, carry):
            sv, sp = sort16(cand_v[pl.ds(rbase + j * 16, 16)],
                            iota + j * 16)
            return merge_top16(carry[0], carry[1], sv, sp)

        return lax.fori_loop(1, nvr, rbody, (bv, bp))

    def select_top32(r8):
        rbase = r8 * CAP
        cnt = cnt_s[r8]
        cand_v[pl.ds(rbase + cnt, 16)] = neg16
        nvr = cnt // 16 + 1
        v1, p1 = round_top16(rbase, nvr)
        saved = plsc.load_gather(cand_v, [rbase + p1])
        plsc.store_scatter(cand_v, [rbase + p1], neg16)
        v2, p2 = round_top16(rbase, nvr)
        plsc.store_scatter(cand_v, [rbase + p1], saved)
        i1 = plsc.load_gather(cand_i, [rbase + p1])
        i2 = plsc.load_gather(cand_i, [rbase + p2])
        return v1, i1, v2, i2

    def append(r8, vals, cols, mask):
        cnt = r8 * CAP + cnt_s[r8]
        plsc.store_compressed(cand_v.at[pl.ds(cnt, 16)], vals, mask=mask)
        plsc.store_compressed(cand_i.at[pl.ds(cnt, 16)], cols, mask=mask)
        cnt_s[r8] = cnt_s[r8] + jnp.sum(mask.astype(jnp.int32))

    def compact(r8):
        @pl.when(cnt_s[r8] > TRIG)
        def _():
            rbase = r8 * CAP
            v1, i1, v2, i2 = select_top32(r8)
            cand_v[pl.ds(rbase, 16)] = v1
            cand_v[pl.ds(rbase + 16, 16)] = v2
            cand_i[pl.ds(rbase, 16)] = i1
            cand_i[pl.ds(rbase + 16, 16)] = i2
            cnt_s[r8] = KP
            thr_v[r8, :] = plsc.load_gather(
                cand_v, [jnp.full((16,), 31, jnp.int32) + rbase])

    def scan_rows(s, cbase, ngroups):
        def row_scan(r8, _):
            def group_body(g, _):
                base = g * GE
                thr = thr_v[r8, :]
                colbase = cbase + base
                vs, ms, pcs, cts = [], [], [], []
                for j in range(G):
                    v = chunk_v[s, r8, pl.ds(base + j * 16, 16)]
                    m = v > thr
                    vs.append(v)
                    ms.append(m)
                    pcs.append(plsc.cumsum(m.astype(jnp.int32)))
                    cts.append(plsc.all_reduce_population_count(m))
                off = jnp.full((16,), r8 * CAP + cnt_s[r8] - 1, jnp.int32)
                offs = []
                for j in range(G):
                    offs.append(off)
                    off = off + cts[j]
                for j in range(G):
                    pos = offs[j] + pcs[j]
                    plsc.store_scatter(cand_v, [pos], vs[j], mask=ms[j])
                    plsc.store_scatter(cand_i, [pos],
                                       colbase + j * 16 + iota, mask=ms[j])
                cnt_s[r8] = jnp.max(off) + 1 - r8 * CAP
                compact(r8)
                return 0

            lax.fori_loop(0, ngroups, group_body, 0)
            return 0

        lax.fori_loop(0, 8, row_scan, 0)

    def octet_body(o, _):
        row8 = pl.multiple_of(row0 + o * 8, 8)

        def full_copy(c, s):
            return pltpu.make_async_copy(
                tgt_hbm.at[pl.ds(row8, 8),
                           pl.ds(pl.multiple_of(c * CHW, 128), CHW)],
                chunk_v.at[s, :, pl.ds(0, CHW)], sem)

        def last_copy(s):
            return pltpu.make_async_copy(
                tgt_hbm.at[pl.ds(row8, 8), pl.ds(NFULL * CHW, LASTW)],
                chunk_v.at[s, :, pl.ds(0, LASTW)], sem)

        def rbody(r8, _):
            thr_v[r8, :] = neg16
            cnt_s[r8] = 0
            return 0

        lax.fori_loop(0, 8, rbody, 0)

        full_copy(0, 0).start()
        pltpu.sync_copy(tail_hbm.at[pl.ds(row8 * TW, 8 * TW)], tail_v)

        def chunk_body(c, _):
            s = lax.rem(c, 2)
            full_copy(c, s).wait()

            @pl.when(c + 1 < NFULL)
            def _():
                full_copy(c + 1, 1 - s).start()

            @pl.when(c + 1 == NFULL)
            def _():
                last_copy(1 - s).start()

            scan_rows(s, c * CHW, NGF)
            return 0

        lax.fori_loop(0, NFULL, chunk_body, 0)
        st = NFULL % 2
        last_copy(st).wait()
        scan_rows(st, NFULL * CHW, NGL)

        def tail_scan(r8, _):
            v0 = tail_v[pl.ds(r8 * TW, 16)]
            v1 = tail_v[pl.ds(r8 * TW + 16, 16)]
            thr = thr_v[r8, :]
            hit = jnp.any(jnp.maximum(v0, v1) > thr)

            @pl.when(hit)
            def _():
                append(r8, v0, NMAIN + iota, v0 > thr)
                append(r8, v1, NMAIN + 16 + iota, v1 > thr)
                compact(r8)

            return 0

        lax.fori_loop(0, 8, tail_scan, 0)

        def out_body(r8, _):
            v1, i1, v2, i2 = select_top32(r8)
            rl = o * 8 + r8
            val_v[pl.ds(rl * KP, 16)] = v1
            val_v[pl.ds(rl * KP + 16, 16)] = v2
            row = row8 + r8
            idx_v[pl.ds(rl * KP, 16)] = _flat_index(row, i1)
            idx_v[pl.ds(rl * KP + 16, 16)] = _flat_index(row, i2)
            return 0

        lax.fori_loop(0, 8, out_body, 0)
        return 0

    lax.fori_loop(0, NOCT, octet_body, 0)

    out0 = wid * RPW * KP
    pltpu.sync_copy(val_v, tt_hbm.at[pl.ds(out0, RPW * KP)])
    pltpu.sync_copy(idx_v, ti_hbm.at[pl.ds(out0, RPW * KP)])


_sc_topk = functools.partial(
    pl.kernel,
    out_type=(jax.ShapeDtypeStruct((B * KP,), jnp.float32),
              jax.ShapeDtypeStruct((B * KP,), jnp.int32)),
    mesh=plsc.VectorSubcoreMesh(core_axis_name="c", subcore_axis_name="s"),
    compiler_params=pltpu.CompilerParams(needs_layout_passes=False),
    scratch_types=(
        pltpu.VMEM((2, 8, CHW), jnp.float32),
        pltpu.VMEM((8 * TW,), jnp.float32),
        pltpu.VMEM((8 * CAP,), jnp.float32),
        pltpu.VMEM((8 * CAP,), jnp.int32),
        pltpu.VMEM((RPW * KP,), jnp.float32),
        pltpu.VMEM((RPW * KP,), jnp.int32),
        pltpu.VMEM((8, 16), jnp.float32),
        pltpu.SMEM((8,), jnp.int32),
        pltpu.SemaphoreType.DMA,
    ),
)(_sc_topk_body)


def _sc_gather_body(lin_hbm, ti_hbm, tl_hbm, idx_v, lgt_v, sem, gsem):
    wid = lax.axis_index("s") * NC + lax.axis_index("c")
    base = wid * RPW * KP
    pltpu.async_copy(ti_hbm.at[pl.ds(base, RPW * KP)], idx_v, sem).wait()
    for q in range(RPW * KP // 128):
        pltpu.async_copy(lin_hbm.at[idx_v.at[pl.ds(q * 128, 128)]],
                         lgt_v.at[pl.ds(q * 128, 128)], gsem).wait()
    pltpu.sync_copy(lgt_v, tl_hbm.at[pl.ds(base, RPW * KP)])


_sc_gather = functools.partial(
    pl.kernel,
    out_type=jax.ShapeDtypeStruct((B * KP,), jnp.float32),
    mesh=plsc.VectorSubcoreMesh(core_axis_name="c", subcore_axis_name="s"),
    compiler_params=pltpu.CompilerParams(needs_layout_passes=False),
    scratch_types=(
        pltpu.VMEM((RPW * KP,), jnp.int32),
        pltpu.VMEM((RPW * KP,), jnp.float32),
        pltpu.SemaphoreType.DMA,
        pltpu.SemaphoreType.DMA,
    ),
)(_sc_gather_body)


TCPB = 4
NLB = (NTC + TCPB - 1) // TCPB


def _lin_body(x_ref, o_ref):
    for t in range(TCPB):
        o_ref[pl.ds(t * TCW, TCW)] = x_ref[:, pl.ds(t * 128, 128)].reshape(TCW)


_lin_logits = pl.pallas_call(
    _lin_body,
    grid=(NLB,),
    in_specs=[pl.BlockSpec((B, 128 * TCPB), lambda i: (0, i))],
    out_specs=pl.BlockSpec((TCPB * TCW,), lambda i: (i,)),
    out_shape=jax.ShapeDtypeStruct((NLB * TCPB * TCW,), jnp.float32),
)


def _loss_body(t_ref, l_ref, o_ref):
    t = t_ref[...]
    l = l_ref[...]
    mask = lax.broadcasted_iota(jnp.int32, (B, KP), 1) < K
    t = jnp.where(mask, t, NEG)
    l = jnp.where(mask, l, NEG)
    tm = jnp.max(t, axis=1, keepdims=True)
    lm = jnp.max(l, axis=1, keepdims=True)
    te = jnp.exp(t - tm)
    le = jnp.exp(l - lm)
    ts = jnp.sum(jnp.where(mask, te, 0.0), axis=1, keepdims=True)
    ls = jnp.sum(jnp.where(mask, le, 0.0), axis=1, keepdims=True)
    pt = te / ts
    diff = (t - tm) - jnp.log(ts) - (l - lm) + jnp.log(ls)
    pw = jnp.where(mask, pt * diff, 0.0)
    o_ref[...] = jnp.broadcast_to(jnp.sum(pw) * (1.0 / B), (1, 1))


def kernel(logits, targets):
    tail = targets[:, NMAIN:].reshape(-1)
    tt_flat, ti_flat = _sc_topk(targets, tail)
    lin = _lin_logits(logits)
    tl_flat = _sc_gather(lin, ti_flat)
    tt = tt_flat.reshape(B, KP)
    tl = tl_flat.reshape(B, KP)
    loss = pl.pallas_call(
        _loss_body,
        out_shape=jax.ShapeDtypeStruct((1, 1), jnp.float32),
    )(tt, tl)
    return loss[0, 0]


# speedup vs baseline: 3.2895x; 1.0160x over previous
import functools

import jax
import jax.numpy as jnp
from jax import lax
from jax.experimental import pallas as pl
from jax.experimental.pallas import tpu as pltpu
from jax.experimental.pallas import tpu_sc as plsc

B = 1024
N = 100000
K = 30
KP = 32
NEG = -1e30
CAP = 1024
G = 22
GE = G * 16
TRIG = CAP - GE - 16
NMAIN = 99968
TW = N - NMAIN
CHW = 5632
NFULL = 17
LASTW = NMAIN - NFULL * CHW
NGF = CHW // GE
NGL = LASTW // GE
NC = 2
NS = 16
NW = NC * NS
RPW = B // NW
NOCT = RPW // 8


NTC = 782
TCW = B * 128


def _flat_index(row, col):
    return (col >> 7) * TCW + row * 128 + (col & 127)


def _sc_topk_body(tgt_hbm, tail_hbm, tt_hbm, ti_hbm,
                  chunk_v, tail_v, cand_v, cand_i, val_v, idx_v,
                  thr_v, cnt_s, sem):
    wid = lax.axis_index("s") * NC + lax.axis_index("c")
    row0 = wid * RPW
    iota = lax.iota(jnp.int32, 16)
    neg16 = jnp.full((16,), NEG, jnp.float32)

    def sort16(v, p):
        sv, sp = plsc.sort_key_val(v, p, descending=True)
        return sv, sp

    def merge_top16(av, ap, bv, bp):
        bvr = lax.rev(bv, (0,))
        bpr = lax.rev(bp, (0,))
        ta = av >= bvr
        return sort16(jnp.where(ta, av, bvr), jnp.where(ta, ap, bpr))

    def round_top16(rbase, nvr):
        bv, bp = sort16(cand_v[pl.ds(rbase, 16)], iota)

        def rbody(j, carry):
            sv, sp = sort16(cand_v[pl.ds(rbase + j * 16, 16)],
                            iota + j * 16)
            return merge_top16(carry[0], carry[1], sv, sp)

        return lax.fori_loop(1, nvr, rbody, (bv, bp))

    def select_top32(r8):
        rbase = r8 * CAP
        cnt = cnt_s[r8]
        cand_v[pl.ds(rbase + cnt, 16)] = neg16
        nvr = cnt // 16 + 1
        v1, p1 = round_top16(rbase, nvr)
        saved = plsc.load_gather(cand_v, [rbase + p1])
        plsc.store_scatter(cand_v, [rbase + p1], neg16)
        v2, p2 = round_top16(rbase, nvr)
        plsc.store_scatter(cand_v, [rbase + p1], saved)
        i1 = plsc.load_gather(cand_i, [rbase + p1])
        i2 = plsc.load_gather(cand_i, [rbase + p2])
        return v1, i1, v2, i2

    def append(r8, vals, cols, mask):
        cnt = r8 * CAP + cnt_s[r8]
        plsc.store_compressed(cand_v.at[pl.ds(cnt, 16)], vals, mask=mask)
        plsc.store_compressed(cand_i.at[pl.ds(cnt, 16)], cols, mask=mask)
        cnt_s[r8] = cnt_s[r8] + jnp.sum(mask.astype(jnp.int32))

    def compact(r8):
        @pl.when(cnt_s[r8] > TRIG)
        def _():
            rbase = r8 * CAP
            v1, i1, v2, i2 = select_top32(r8)
            cand_v[pl.ds(rbase, 16)] = v1
            cand_v[pl.ds(rbase + 16, 16)] = v2
            cand_i[pl.ds(rbase, 16)] = i1
            cand_i[pl.ds(rbase + 16, 16)] = i2
            cnt_s[r8] = KP
            thr_v[r8, :] = plsc.load_gather(
                cand_v, [jnp.full((16,), 31, jnp.int32) + rbase])

    def scan_rows(s, cbase, ngroups):
        def row_scan(r8, _):
            def group_body(g, _):
                base = g * GE
                thr = thr_v[r8, :]
                colbase = cbase + base
                vs, ms, pcs, cts = [], [], [], []
                for j in range(G):
                    v = chunk_v[s, r8, pl.ds(base + j * 16, 16)]
                    m = v > thr
                    vs.append(v)
                    ms.append(m)
                    pcs.append(plsc.cumsum(m.astype(jnp.int32)))
                    cts.append(plsc.all_reduce_population_count(m))
                off = jnp.full((16,), r8 * CAP + cnt_s[r8] - 1, jnp.int32)
                offs = []
                for j in range(G):
                    offs.append(off)
                    off = off + cts[j]
                for j in range(G):
                    pos = offs[j] + pcs[j]
                    plsc.store_scatter(cand_v, [pos], vs[j], mask=ms[j])
                    plsc.store_scatter(cand_i, [pos],
                                       colbase + j * 16 + iota, mask=ms[j])
                cnt_s[r8] = jnp.max(off) + 1 - r8 * CAP
                compact(r8)
                return 0

            lax.fori_loop(0, ngroups, group_body, 0)
            return 0

        lax.fori_loop(0, 8, row_scan, 0)

    def octet_body(o, _):
        row8 = pl.multiple_of(row0 + o * 8, 8)

        def full_copy(c, s):
            return pltpu.make_async_copy(
                tgt_hbm.at[pl.ds(row8, 8),
                           pl.ds(pl.multiple_of(c * CHW, 128), CHW)],
                chunk_v.at[s, :, pl.ds(0, CHW)], sem)

        def last_copy(s):
            return pltpu.make_async_copy(
                tgt_hbm.at[pl.ds(row8, 8), pl.ds(NFULL * CHW, LASTW)],
                chunk_v.at[s, :, pl.ds(0, LASTW)], sem)

        def rbody(r8, _):
            thr_v[r8, :] = neg16
            cnt_s[r8] = 0
            return 0

        lax.fori_loop(0, 8, rbody, 0)

        full_copy(0, 0).start()
        pltpu.sync_copy(tail_hbm.at[pl.ds(row8 * TW, 8 * TW)], tail_v)

        def chunk_body(c, _):
            s = lax.rem(c, 2)
            full_copy(c, s).wait()

            @pl.when(c + 1 < NFULL)
            def _():
                full_copy(c + 1, 1 - s).start()

            @pl.when(c + 1 == NFULL)
            def _():
                last_copy(1 - s).start()

            scan_rows(s, c * CHW, NGF)
            return 0

        lax.fori_loop(0, NFULL, chunk_body, 0)
        st = NFULL % 2
        last_copy(st).wait()
        scan_rows(st, NFULL * CHW, NGL)

        def tail_scan(r8, _):
            v0 = tail_v[pl.ds(r8 * TW, 16)]
            v1 = tail_v[pl.ds(r8 * TW + 16, 16)]
            thr = thr_v[r8, :]
            hit = jnp.any(jnp.maximum(v0, v1) > thr)

            @pl.when(hit)
            def _():
                append(r8, v0, NMAIN + iota, v0 > thr)
                append(r8, v1, NMAIN + 16 + iota, v1 > thr)
                compact(r8)

            return 0

        lax.fori_loop(0, 8, tail_scan, 0)

        def out_body(r8, _):
            v1, i1, v2, i2 = select_top32(r8)
            rl = o * 8 + r8
            val_v[pl.ds(rl * KP, 16)] = v1
            val_v[pl.ds(rl * KP + 16, 16)] = v2
            row = row8 + r8
            idx_v[pl.ds(rl * KP, 16)] = _flat_index(row, i1)
            idx_v[pl.ds(rl * KP + 16, 16)] = _flat_index(row, i2)
            return 0

        lax.fori_loop(0, 8, out_body, 0)
        return 0

    lax.fori_loop(0, NOCT, octet_body, 0)

    out0 = wid * RPW * KP
    pltpu.sync_copy(val_v, tt_hbm.at[pl.ds(out0, RPW * KP)])
    pltpu.sync_copy(idx_v, ti_hbm.at[pl.ds(out0, RPW * KP)])


_sc_topk = functools.partial(
    pl.kernel,
    out_type=(jax.ShapeDtypeStruct((B * KP,), jnp.float32),
              jax.ShapeDtypeStruct((B * KP,), jnp.int32)),
    mesh=plsc.VectorSubcoreMesh(core_axis_name="c", subcore_axis_name="s"),
    compiler_params=pltpu.CompilerParams(needs_layout_passes=False),
    scratch_types=(
        pltpu.VMEM((2, 8, CHW), jnp.float32),
        pltpu.VMEM((8 * TW,), jnp.float32),
        pltpu.VMEM((8 * CAP,), jnp.float32),
        pltpu.VMEM((8 * CAP,), jnp.int32),
        pltpu.VMEM((RPW * KP,), jnp.float32),
        pltpu.VMEM((RPW * KP,), jnp.int32),
        pltpu.VMEM((8, 16), jnp.float32),
        pltpu.SMEM((8,), jnp.int32),
        pltpu.SemaphoreType.DMA,
    ),
)(_sc_topk_body)


def _sc_gather_body(lin_hbm, ti_hbm, tl_hbm, idx_v, lgt_v, sem, gsem):
    wid = lax.axis_index("s") * NC + lax.axis_index("c")
    base = wid * RPW * KP
    pltpu.async_copy(ti_hbm.at[pl.ds(base, RPW * KP)], idx_v, sem).wait()
    for q in range(RPW * KP // 128):
        pltpu.async_copy(lin_hbm.at[idx_v.at[pl.ds(q * 128, 128)]],
                         lgt_v.at[pl.ds(q * 128, 128)], gsem).wait()
    pltpu.sync_copy(lgt_v, tl_hbm.at[pl.ds(base, RPW * KP)])


_sc_gather = functools.partial(
    pl.kernel,
    out_type=jax.ShapeDtypeStruct((B * KP,), jnp.float32),
    mesh=plsc.VectorSubcoreMesh(core_axis_name="c", subcore_axis_name="s"),
    compiler_params=pltpu.CompilerParams(needs_layout_passes=False),
    scratch_types=(
        pltpu.VMEM((RPW * KP,), jnp.int32),
        pltpu.VMEM((RPW * KP,), jnp.float32),
        pltpu.SemaphoreType.DMA,
        pltpu.SemaphoreType.DMA,
    ),
)(_sc_gather_body)


TCPB = 8
NLB = (NTC + TCPB - 1) // TCPB


def _lin_body(x_ref, o_ref):
    for t in range(TCPB):
        o_ref[pl.ds(t * TCW, TCW)] = x_ref[:, pl.ds(t * 128, 128)].reshape(TCW)


_lin_logits = pl.pallas_call(
    _lin_body,
    grid=(NLB,),
    in_specs=[pl.BlockSpec((B, 128 * TCPB), lambda i: (0, i))],
    out_specs=pl.BlockSpec((TCPB * TCW,), lambda i: (i,)),
    out_shape=jax.ShapeDtypeStruct((NLB * TCPB * TCW,), jnp.float32),
)


def _loss_body(t_ref, l_ref, o_ref):
    t = t_ref[...]
    l = l_ref[...]
    mask = lax.broadcasted_iota(jnp.int32, (B, KP), 1) < K
    t = jnp.where(mask, t, NEG)
    l = jnp.where(mask, l, NEG)
    tm = jnp.max(t, axis=1, keepdims=True)
    lm = jnp.max(l, axis=1, keepdims=True)
    te = jnp.exp(t - tm)
    le = jnp.exp(l - lm)
    ts = jnp.sum(jnp.where(mask, te, 0.0), axis=1, keepdims=True)
    ls = jnp.sum(jnp.where(mask, le, 0.0), axis=1, keepdims=True)
    pt = te / ts
    diff = (t - tm) - jnp.log(ts) - (l - lm) + jnp.log(ls)
    pw = jnp.where(mask, pt * diff, 0.0)
    o_ref[...] = jnp.broadcast_to(jnp.sum(pw) * (1.0 / B), (1, 1))


def kernel(logits, targets):
    tail = targets[:, NMAIN:].reshape(-1)
    tt_flat, ti_flat = _sc_topk(targets, tail)
    lin = _lin_logits(logits)
    tl_flat = _sc_gather(lin, ti_flat)
    tt = tt_flat.reshape(B, KP)
    tl = tl_flat.reshape(B, KP)
    loss = pl.pallas_call(
        _loss_body,
        out_shape=jax.ShapeDtypeStruct((1, 1), jnp.float32),
    )(tt, tl)
    return loss[0, 0]


# speedup vs baseline: 3.2983x; 1.0027x over previous
import functools

import jax
import jax.numpy as jnp
from jax import lax
from jax.experimental import pallas as pl
from jax.experimental.pallas import tpu as pltpu
from jax.experimental.pallas import tpu_sc as plsc

B = 1024
N = 100000
K = 30
KP = 32
NEG = -1e30
CAP = 1024
G = 22
GE = G * 16
TRIG = CAP - GE - 16
NMAIN = 99968
TW = N - NMAIN
CHW = 5632
NFULL = 17
LASTW = NMAIN - NFULL * CHW
NGF = CHW // GE
NGL = LASTW // GE
NC = 2
NS = 16
NW = NC * NS
RPW = B // NW
NOCT = RPW // 8


NTC = 782
TCW = B * 128


def _flat_index(row, col):
    return (col >> 7) * TCW + row * 128 + (col & 127)


def _sc_topk_body(tgt_hbm, tail_hbm, tt_hbm, ti_hbm,
                  chunk_v, tail_v, cand_v, cand_i, val_v, idx_v,
                  thr_v, cnt_s, sem):
    wid = lax.axis_index("s") * NC + lax.axis_index("c")
    row0 = wid * RPW
    iota = lax.iota(jnp.int32, 16)
    neg16 = jnp.full((16,), NEG, jnp.float32)

    def sort16(v, p):
        sv, sp = plsc.sort_key_val(v, p, descending=True)
        return sv, sp

    def merge_top16(av, ap, bv, bp):
        bvr = lax.rev(bv, (0,))
        bpr = lax.rev(bp, (0,))
        ta = av >= bvr
        return sort16(jnp.where(ta, av, bvr), jnp.where(ta, ap, bpr))

    def round_top16(rbase, nvr):
        bv, bp = sort16(cand_v[pl.ds(rbase, 16)], iota)

        def rbody(j, carry):
            sv, sp = sort16(cand_v[pl.ds(rbase + j * 16, 16)],
                            iota + j * 16)
            return merge_top16(carry[0], carry[1], sv, sp)

        return lax.fori_loop(1, nvr, rbody, (bv, bp))

    def select_top32(r8):
        rbase = r8 * CAP
        cnt = cnt_s[r8]
        cand_v[pl.ds(rbase + cnt, 16)] = neg16
        nvr = cnt // 16 + 1
        v1, p1 = round_top16(rbase, nvr)
        saved = plsc.load_gather(cand_v, [rbase + p1])
        plsc.store_scatter(cand_v, [rbase + p1], neg16)
        v2, p2 = round_top16(rbase, nvr)
        plsc.store_scatter(cand_v, [rbase + p1], saved)
        i1 = plsc.load_gather(cand_i, [rbase + p1])
        i2 = plsc.load_gather(cand_i, [rbase + p2])
        return v1, i1, v2, i2

    def append(r8, vals, cols, mask):
        cnt = r8 * CAP + cnt_s[r8]
        plsc.store_compressed(cand_v.at[pl.ds(cnt, 16)], vals, mask=mask)
        plsc.store_compressed(cand_i.at[pl.ds(cnt, 16)], cols, mask=mask)
        cnt_s[r8] = cnt_s[r8] + jnp.sum(mask.astype(jnp.int32))

    def compact(r8):
        @pl.when(cnt_s[r8] > TRIG)
        def _():
            rbase = r8 * CAP
            v1, i1, v2, i2 = select_top32(r8)
            cand_v[pl.ds(rbase, 16)] = v1
            cand_v[pl.ds(rbase + 16, 16)] = v2
            cand_i[pl.ds(rbase, 16)] = i1
            cand_i[pl.ds(rbase + 16, 16)] = i2
            cnt_s[r8] = KP
            thr_v[r8, :] = plsc.load_gather(
                cand_v, [jnp.full((16,), 31, jnp.int32) + rbase])

    def scan_rows(s, cbase, ngroups):
        def row_scan(r8, _):
            def group_body(g, _):
                base = g * GE
                thr = thr_v[r8, :]
                colbase = cbase + base
                vs, ms, pcs, cts = [], [], [], []
                for j in range(G):
                    v = chunk_v[s, r8, pl.ds(base + j * 16, 16)]
                    m = v > thr
                    vs.append(v)
                    ms.append(m)
                    pcs.append(plsc.cumsum(m.astype(jnp.int32)))
                    cts.append(plsc.all_reduce_population_count(m))
                off = jnp.full((16,), r8 * CAP + cnt_s[r8] - 1, jnp.int32)
                offs = []
                for j in range(G):
                    offs.append(off)
                    off = off + cts[j]
                for j in range(G):
                    pos = offs[j] + pcs[j]
                    plsc.store_scatter(cand_v, [pos], vs[j], mask=ms[j])
                    plsc.store_scatter(cand_i, [pos],
                                       colbase + j * 16 + iota, mask=ms[j])
                cnt_s[r8] = jnp.max(off) + 1 - r8 * CAP
                compact(r8)
                return 0

            lax.fori_loop(0, ngroups, group_body, 0)
            return 0

        lax.fori_loop(0, 8, row_scan, 0)

    def octet_body(o, _):
        row8 = pl.multiple_of(row0 + o * 8, 8)

        def full_copy(c, s):
            return pltpu.make_async_copy(
                tgt_hbm.at[pl.ds(row8, 8),
                           pl.ds(pl.multiple_of(c * CHW, 128), CHW)],
                chunk_v.at[s, :, pl.ds(0, CHW)], sem)

        def last_copy(s):
            return pltpu.make_async_copy(
                tgt_hbm.at[pl.ds(row8, 8), pl.ds(NFULL * CHW, LASTW)],
                chunk_v.at[s, :, pl.ds(0, LASTW)], sem)

        def rbody(r8, _):
            thr_v[r8, :] = neg16
            cnt_s[r8] = 0
            return 0

        lax.fori_loop(0, 8, rbody, 0)

        full_copy(0, 0).start()
        pltpu.sync_copy(tail_hbm.at[pl.ds(row8 * TW, 8 * TW)], tail_v)

        def chunk_body(c, _):
            s = lax.rem(c, 2)
            full_copy(c, s).wait()

            @pl.when(c + 1 < NFULL)
            def _():
                full_copy(c + 1, 1 - s).start()

            @pl.when(c + 1 == NFULL)
            def _():
                last_copy(1 - s).start()

            scan_rows(s, c * CHW, NGF)
            return 0

        lax.fori_loop(0, NFULL, chunk_body, 0)
        st = NFULL % 2
        last_copy(st).wait()
        scan_rows(st, NFULL * CHW, NGL)

        def tail_scan(r8, _):
            v0 = tail_v[pl.ds(r8 * TW, 16)]
            v1 = tail_v[pl.ds(r8 * TW + 16, 16)]
            thr = thr_v[r8, :]
            hit = jnp.any(jnp.maximum(v0, v1) > thr)

            @pl.when(hit)
            def _():
                append(r8, v0, NMAIN + iota, v0 > thr)
                append(r8, v1, NMAIN + 16 + iota, v1 > thr)
                compact(r8)

            return 0

        lax.fori_loop(0, 8, tail_scan, 0)

        def out_body(r8, _):
            v1, i1, v2, i2 = select_top32(r8)
            rl = o * 8 + r8
            val_v[pl.ds(rl * KP, 16)] = v1
            val_v[pl.ds(rl * KP + 16, 16)] = v2
            row = row8 + r8
            idx_v[pl.ds(rl * KP, 16)] = _flat_index(row, i1)
            idx_v[pl.ds(rl * KP + 16, 16)] = _flat_index(row, i2)
            return 0

        lax.fori_loop(0, 8, out_body, 0)
        return 0

    lax.fori_loop(0, NOCT, octet_body, 0)

    out0 = wid * RPW * KP
    pltpu.sync_copy(val_v, tt_hbm.at[pl.ds(out0, RPW * KP)])
    pltpu.sync_copy(idx_v, ti_hbm.at[pl.ds(out0, RPW * KP)])


_sc_topk = functools.partial(
    pl.kernel,
    out_type=(jax.ShapeDtypeStruct((B * KP,), jnp.float32),
              jax.ShapeDtypeStruct((B * KP,), jnp.int32)),
    mesh=plsc.VectorSubcoreMesh(core_axis_name="c", subcore_axis_name="s"),
    compiler_params=pltpu.CompilerParams(needs_layout_passes=False),
    scratch_types=(
        pltpu.VMEM((2, 8, CHW), jnp.float32),
        pltpu.VMEM((8 * TW,), jnp.float32),
        pltpu.VMEM((8 * CAP,), jnp.float32),
        pltpu.VMEM((8 * CAP,), jnp.int32),
        pltpu.VMEM((RPW * KP,), jnp.float32),
        pltpu.VMEM((RPW * KP,), jnp.int32),
        pltpu.VMEM((8, 16), jnp.float32),
        pltpu.SMEM((8,), jnp.int32),
        pltpu.SemaphoreType.DMA,
    ),
)(_sc_topk_body)


def _sc_gather_body(lin_hbm, ti_hbm, tl_hbm, idx_v, lgt_v, sem, gsem):
    wid = lax.axis_index("s") * NC + lax.axis_index("c")
    base = wid * RPW * KP
    pltpu.async_copy(ti_hbm.at[pl.ds(base, RPW * KP)], idx_v, sem).wait()
    for q in range(RPW * KP // 128):
        pltpu.async_copy(lin_hbm.at[idx_v.at[pl.ds(q * 128, 128)]],
                         lgt_v.at[pl.ds(q * 128, 128)], gsem).wait()
    pltpu.sync_copy(lgt_v, tl_hbm.at[pl.ds(base, RPW * KP)])


_sc_gather = functools.partial(
    pl.kernel,
    out_type=jax.ShapeDtypeStruct((B * KP,), jnp.float32),
    mesh=plsc.VectorSubcoreMesh(core_axis_name="c", subcore_axis_name="s"),
    compiler_params=pltpu.CompilerParams(needs_layout_passes=False),
    scratch_types=(
        pltpu.VMEM((RPW * KP,), jnp.int32),
        pltpu.VMEM((RPW * KP,), jnp.float32),
        pltpu.SemaphoreType.DMA,
        pltpu.SemaphoreType.DMA,
    ),
)(_sc_gather_body)


TCPB = 16
NLB = (NTC + TCPB - 1) // TCPB


def _lin_body(x_ref, o_ref):
    for t in range(TCPB):
        o_ref[pl.ds(t * TCW, TCW)] = x_ref[:, pl.ds(t * 128, 128)].reshape(TCW)


_lin_logits = pl.pallas_call(
    _lin_body,
    grid=(NLB,),
    in_specs=[pl.BlockSpec((B, 128 * TCPB), lambda i: (0, i))],
    out_specs=pl.BlockSpec((TCPB * TCW,), lambda i: (i,)),
    out_shape=jax.ShapeDtypeStruct((NLB * TCPB * TCW,), jnp.float32),
)


def _loss_body(t_ref, l_ref, o_ref):
    t = t_ref[...]
    l = l_ref[...]
    mask = lax.broadcasted_iota(jnp.int32, (B, KP), 1) < K
    t = jnp.where(mask, t, NEG)
    l = jnp.where(mask, l, NEG)
    tm = jnp.max(t, axis=1, keepdims=True)
    lm = jnp.max(l, axis=1, keepdims=True)
    te = jnp.exp(t - tm)
    le = jnp.exp(l - lm)
    ts = jnp.sum(jnp.where(mask, te, 0.0), axis=1, keepdims=True)
    ls = jnp.sum(jnp.where(mask, le, 0.0), axis=1, keepdims=True)
    pt = te / ts
    diff = (t - tm) - jnp.log(ts) - (l - lm) + jnp.log(ls)
    pw = jnp.where(mask, pt * diff, 0.0)
    o_ref[...] = jnp.broadcast_to(jnp.sum(pw) * (1.0 / B), (1, 1))


def kernel(logits, targets):
    tail = targets[:, NMAIN:].reshape(-1)
    tt_flat, ti_flat = _sc_topk(targets, tail)
    lin = _lin_logits(logits)
    tl_flat = _sc_gather(lin, ti_flat)
    tt = tt_flat.reshape(B, KP)
    tl = tl_flat.reshape(B, KP)
    loss = pl.pallas_call(
        _loss_body,
        out_shape=jax.ShapeDtypeStruct((1, 1), jnp.float32),
    )(tt, tl)
    return loss[0, 0]
